# baseline (speedup 1.0000x reference)
import functools

import jax
import jax.numpy as jnp
from jax import lax
from jax.experimental import pallas as pl
from jax.experimental.pallas import tpu as pltpu
from jax.experimental.pallas import tpu_sc as plsc

N = 10000
E = 320000
F_IN = 128
H = 256

NP = 10240
E2 = 327680
ER = E2 // 128
CHUNKS = 40
DUMP = 10008
SELF = 10000
ACC_R = 10112
STRIPE = 632
NB = 10000 // 512 + 1
BLK = 512

_mesh = functools.partial(
    plsc.VectorSubcoreMesh, core_axis_name="c", subcore_axis_name="s")


def _rsqrt16(x):
    i = lax.bitcast_convert_type(x, jnp.int32)
    y = lax.bitcast_convert_type(jnp.int32(0x5F3759DF) - (i >> 1), jnp.float32)
    for _ in range(4):
        y = y * (1.5 - 0.5 * x * y * y)
    return y


def _p12_body(srcp, dstp, s_out, d2_out, srcs_out,
              sidxall, didxall, ones, gs0, gs1, gd0, gd1, nrm0, nrm1,
              sbuf, work, work2, dinv_sh, acc, sem_g, sem_w):
    c = lax.axis_index("c")
    s = lax.axis_index("s")
    rt0 = s * 160
    for k in range(8):
        ones[pl.ds(k * 16, 16)] = jnp.full((16,), 1.0, jnp.float32)
    for k in range(40):
        work[pl.ds(k * 16, 16)] = jnp.zeros((16,), jnp.float32)
    pltpu.sync_copy(work, acc.at[pl.ds(s * 640, 640)])
    pltpu.sync_copy(srcp.at[pl.ds(rt0, 160)], sidxall)
    pltpu.sync_copy(dstp.at[pl.ds(rt0, 160)], didxall)
    plsc.subcore_barrier()

    def dchunk(i, carry):
        for j in range(4):
            pltpu.sync_copy(ones, acc.at[sidxall.at[i * 4 + j]], add=True)
        return carry

    lax.fori_loop(0, 40, dchunk, 0)
    plsc.subcore_barrier()
    pltpu.sync_copy(acc.at[pl.ds(s * 640, 640)], work)
    for k in range(40):
        o = pl.ds(k * 16, 16)
        work[o] = _rsqrt16(work[o] + 1.0)
        work2[o] = jnp.zeros((16,), jnp.float32)
    pltpu.sync_copy(work, dinv_sh.at[pl.ds(s * 640, 640)])
    pltpu.sync_copy(work2, acc.at[pl.ds(s * 640, 640)])
    plsc.subcore_barrier()

    def issue(q, gs, gd):
        pltpu.async_copy(dinv_sh.at[sidxall.at[q]], gs, sem_g)
        pltpu.async_copy(dinv_sh.at[didxall.at[q]], gd, sem_g)

    def gwait(q, gs, gd):
        pltpu.make_async_copy(dinv_sh.at[sidxall.at[q]], gs, sem_g).wait()
        pltpu.make_async_copy(dinv_sh.at[didxall.at[q]], gd, sem_g).wait()

    bufs = [(gs0, gd0, nrm0), (gs1, gd1, nrm1)]
    issue(0, gs0, gd0)

    def eblk(k, carry):
        @pl.when(k > 0)
        def _():
            pltpu.make_async_copy(
                sbuf, srcs_out.at[pl.ds(c * ER + rt0, 8)], sem_w).wait()
        for j in range(8):
            q = k * 8 + j
            gs, gd, nrm = bufs[j % 2]
            gs2, gd2, _ = bufs[(j + 1) % 2]
            if j < 7:
                issue(q + 1, gs2, gd2)
            else:
                @pl.when(k < 19)
                def _():
                    issue(q + 1, gs2, gd2)
            gwait(q, gs, gd)
            for t in range(8):
                o = pl.ds(t * 16, 16)
                sv = sidxall[q, o]
                dv = didxall[q, o]
                eq = sv == dv
                nrm[o] = jnp.where(eq, 0.0, gs[o] * gd[o])
                sbuf[j, o] = (jnp.where(eq, jnp.int32(SELF), sv) + c * NP) | (
                    jnp.where(dv < N, dv, jnp.int32(DUMP)) << 16)
            pltpu.sync_copy(nrm, acc.at[didxall.at[q]], add=True)
        pltpu.async_copy(
            sbuf, srcs_out.at[pl.ds(c * ER + rt0 + k * 8, 8)], sem_w)
        return carry

    lax.fori_loop(0, 20, eblk, 0)
    pltpu.make_async_copy(
        sbuf, srcs_out.at[pl.ds(c * ER + rt0, 8)], sem_w).wait()
    plsc.subcore_barrier()
    base = c * 5120 + s * 320
    pltpu.sync_copy(acc.at[pl.ds(base, 320)], work.at[pl.ds(0, 320)])
    pltpu.sync_copy(dinv_sh.at[pl.ds(base, 320)], work2.at[pl.ds(0, 320)])
    for k in range(20):
        o = pl.ds(k * 16, 16)
        r2 = _rsqrt16(work[o] + 2.0)
        dv = work2[o]
        work[o] = dv * r2
        work2[o] = 2.0 * r2 / dv
    pltpu.sync_copy(work.at[pl.ds(0, 320)], s_out.at[pl.ds(base, 320)])
    pltpu.sync_copy(work2.at[pl.ds(0, 320)], d2_out.at[pl.ds(base, 320)])


_p12 = pl.kernel(
    _p12_body,
    out_type=[
        jax.ShapeDtypeStruct((NP,), jnp.float32),
        jax.ShapeDtypeStruct((NP,), jnp.float32),
        jax.ShapeDtypeStruct((2 * ER, 128), jnp.int32),
    ],
    mesh=_mesh(),
    scratch_types=[
        pltpu.VMEM((160, 128), jnp.int32),
        pltpu.VMEM((160, 128), jnp.int32),
        pltpu.VMEM((128,), jnp.float32),
        pltpu.VMEM((128,), jnp.float32),
        pltpu.VMEM((128,), jnp.float32),
        pltpu.VMEM((128,), jnp.float32),
        pltpu.VMEM((128,), jnp.float32),
        pltpu.VMEM((128,), jnp.float32),
        pltpu.VMEM((128,), jnp.float32),
        pltpu.VMEM((8, 128), jnp.int32),
        pltpu.VMEM((640,), jnp.float32),
        pltpu.VMEM((640,), jnp.float32),
        pltpu.VMEM_SHARED((NP,), jnp.float32),
        pltpu.VMEM_SHARED((NP,), jnp.float32),
        pltpu.SemaphoreType.DMA,
        pltpu.SemaphoreType.DMA,
    ],
)


def _spmm_body(xlp, pidx_hbm, zeros, y_out,
               pidx, suA, duA, suB, duB, rowsA, rowsB, acc, sem):
    c = lax.axis_index("c")
    s = lax.axis_index("s")
    pltpu.sync_copy(zeros, acc.at[pl.ds(s * STRIPE, STRIPE)])
    plsc.subcore_barrier()

    def unpack(q, su, du):
        for k in range(8):
            o = pl.ds(k * 16, 16)
            p16 = pidx[q, o]
            su[k // 2, pl.ds((k % 2) * 16, 16)] = p16 & jnp.int32(0xFFFF)
            du[0, o] = p16 >> 16

    def gather(su, rows):
        for j in range(4):
            pltpu.async_copy(xlp.at[su.at[j]], rows.at[pl.ds(j * 32, 32)], sem)

    def gwait(su, rows):
        for j in range(4):
            pltpu.make_async_copy(
                xlp.at[su.at[j]], rows.at[pl.ds(j * 32, 32)], sem).wait()

    def scat(rows, du):
        pltpu.sync_copy(rows, acc.at[du.at[0]], add=True)

    for p in range(2):
        pltpu.sync_copy(pidx_hbm.at[pl.ds(c * ER + s * 160 + p * 80, 80)], pidx)
        unpack(0, suA, duA)
        gather(suA, rowsA)

        def it(k, carry):
            q = 2 * k
            unpack(q + 1, suB, duB)
            gather(suB, rowsB)
            gwait(suA, rowsA)
            scat(rowsA, duA)
            unpack(q + 2, suA, duA)
            gather(suA, rowsA)
            gwait(suB, rowsB)
            scat(rowsB, duB)
            return carry

        lax.fori_loop(0, 39, it, 0)
        unpack(79, suB, duB)
        gather(suB, rowsB)
        gwait(suA, rowsA)
        scat(rowsA, duA)
        gwait(suB, rowsB)
        scat(rowsB, duB)

    plsc.subcore_barrier()
    pltpu.sync_copy(acc.at[pl.ds(s * STRIPE, STRIPE)],
                    y_out.at[pl.ds(c * NP + s * STRIPE, STRIPE)])


_spmm = pl.kernel(
    _spmm_body,
    out_type=jax.ShapeDtypeStruct((2 * NP, 128), jnp.float32),
    mesh=_mesh(),
    scratch_types=[
        pltpu.VMEM((80, 128), jnp.int32),
        pltpu.VMEM((4, 32), jnp.int32),
        pltpu.VMEM((1, 128), jnp.int32),
        pltpu.VMEM((4, 32), jnp.int32),
        pltpu.VMEM((1, 128), jnp.int32),
        pltpu.VMEM((128, 128), jnp.float32),
        pltpu.VMEM((128, 128), jnp.float32),
        pltpu.VMEM_SHARED((ACC_R, 128), jnp.float32),
        pltpu.SemaphoreType.DMA,
    ],
)


def _lin1_body(x_ref, w1_ref, sw_ref, sb_ref, sc_ref, xlp_ref, res_ref):
    x = x_ref[...]
    sc = sc_ref[...]
    mm = lax.dot_general(x, w1_ref[...], (((1,), (1,)), ((), ())),
                         preferred_element_type=jnp.float32)
    res_ref[...] = lax.dot_general(x, sw_ref[...], (((1,), (1,)), ((), ())),
                                   preferred_element_type=jnp.float32) + sb_ref[...]
    xlp_ref[0] = sc * mm[:, 0:128]
    xlp_ref[1] = sc * mm[:, 128:256]


def _statapply_body(y_ref, xlp_ref, sc_ref, d2_ref, b_ref, g_ref, be_ref,
                    res_ref, w2_ref, h1_ref, xlp2_ref, sums):
    i = pl.program_id(0)
    sc = sc_ref[...]
    d2 = d2_ref[...]
    b = b_ref[...]
    o_l = sc * y_ref[0] + d2 * xlp_ref[0] + b[:, 0:128]
    o_r = sc * y_ref[1] + d2 * xlp_ref[1] + b[:, 128:256]
    out = jnp.concatenate([o_l, o_r], axis=1)
    rows = lax.broadcasted_iota(jnp.int32, (BLK, 1), 0) + (i % NB) * BLK
    mask = rows < N

    @pl.when(i < NB)
    def _():
        mo = jnp.where(mask, out, 0.0)
        ps = jnp.concatenate(
            [jnp.sum(mo, axis=0, keepdims=True),
             jnp.sum(mo * mo, axis=0, keepdims=True)], axis=0)

        @pl.when(i == 0)
        def _():
            sums[...] = ps

        @pl.when(i > 0)
        def _():
            sums[...] += ps

    @pl.when(i >= NB)
    def _():
        mu = sums[0:1] * (1.0 / N)
        var = sums[1:2] * (1.0 / N) - mu * mu
        rs = lax.rsqrt(var + 1e-5)
        bn = g_ref[...] * (out - mu) * rs + be_ref[...]
        h1 = jnp.maximum(jnp.maximum(bn, 0.0) + res_ref[...], 0.0)
        h1 = jnp.where(mask, h1, 0.0)
        h1_ref[...] = h1
        mm = lax.dot_general(h1, w2_ref[...], (((1,), (1,)), ((), ())),
                             preferred_element_type=jnp.float32)
        xlp2_ref[0] = sc * mm[:, 0:128]
        xlp2_ref[1] = sc * mm[:, 128:256]


def _statfinal_body(y_ref, xlp_ref, sc_ref, d2_ref, b_ref, g_ref, be_ref,
                    h1_ref, r_ref, sums):
    i = pl.program_id(0)
    sc = sc_ref[...]
    d2 = d2_ref[...]
    b = b_ref[...]
    o_l = sc * y_ref[0] + d2 * xlp_ref[0] + b[:, 0:128]
    o_r = sc * y_ref[1] + d2 * xlp_ref[1] + b[:, 128:256]
    out = jnp.concatenate([o_l, o_r], axis=1)
    rows = lax.broadcasted_iota(jnp.int32, (BLK, 1), 0) + (i % NB) * BLK
    mask = rows < N

    @pl.when(i < NB)
    def _():
        mo = jnp.where(mask, out, 0.0)
        ps = jnp.concatenate(
            [jnp.sum(mo, axis=0, keepdims=True),
             jnp.sum(mo * mo, axis=0, keepdims=True)], axis=0)

        @pl.when(i == 0)
        def _():
            sums[...] = ps

        @pl.when(i > 0)
        def _():
            sums[...] += ps

    @pl.when(i >= NB)
    def _():
        mu = sums[0:1] * (1.0 / N)
        var = sums[1:2] * (1.0 / N) - mu * mu
        rs = lax.rsqrt(var + 1e-5)
        bn = g_ref[...] * (out - mu) * rs + be_ref[...]
        h2 = jnp.maximum(jnp.maximum(bn, 0.0) + h1_ref[...], 0.0)
        nrm = jnp.sqrt(jnp.sum(h2 * h2, axis=1, keepdims=True))
        r_ref[...] = h2 / jnp.maximum(nrm, 1e-12)


def _row_spec(w):
    return pl.BlockSpec((BLK, w), lambda i: (i, 0))


def _half_spec():
    return pl.BlockSpec((2, BLK, 128), lambda i: (0, i, 0))


def _full_spec(shape):
    return pl.BlockSpec(shape, lambda i: tuple(0 for _ in shape))


_lin1 = pl.pallas_call(
    _lin1_body,
    grid=(NB,),
    in_specs=[_row_spec(128), _full_spec((H, F_IN)), _full_spec((H, F_IN)),
              _full_spec((1, H)), _row_spec(1)],
    out_specs=[_half_spec(), _row_spec(H)],
    out_shape=[jax.ShapeDtypeStruct((2, NP, 128), jnp.float32),
               jax.ShapeDtypeStruct((NP, H), jnp.float32)],
)

_mod_spec = lambda w: pl.BlockSpec((BLK, w), lambda i: (i % NB, 0))
_mod_half = pl.BlockSpec((2, BLK, 128), lambda i: (0, i % NB, 0))

_statapply = pl.pallas_call(
    _statapply_body,
    grid=(2 * NB,),
    in_specs=[_mod_half, _mod_half, _mod_spec(1), _mod_spec(1),
              _full_spec((1, H)), _full_spec((1, H)), _full_spec((1, H)),
              _mod_spec(H), _full_spec((H, H))],
    out_specs=[_mod_spec(H), _mod_half],
    out_shape=[jax.ShapeDtypeStruct((NP, H), jnp.float32),
               jax.ShapeDtypeStruct((2, NP, 128), jnp.float32)],
    scratch_shapes=[pltpu.VMEM((2, H), jnp.float32)],
)

_statfinal = pl.pallas_call(
    _statfinal_body,
    grid=(2 * NB,),
    in_specs=[_mod_half, _mod_half, _mod_spec(1), _mod_spec(1),
              _full_spec((1, H)), _full_spec((1, H)), _full_spec((1, H)),
              _mod_spec(H)],
    out_specs=_mod_spec(H),
    out_shape=jax.ShapeDtypeStruct((NP, H), jnp.float32),
    scratch_shapes=[pltpu.VMEM((2, H), jnp.float32)],
)


def kernel(x, edge_index, W1, b1, g1, be1, sW, sb, W2, b2, g2, be2):
    pad = jnp.full((E2 - E,), DUMP, jnp.int32)
    srcp = jnp.concatenate([edge_index[0], pad]).reshape(ER, 128)
    dstp = jnp.concatenate([edge_index[1], pad]).reshape(ER, 128)
    x_pad = jnp.pad(x, ((0, NP - N), (0, 0)))
    zeros = jnp.zeros((STRIPE, 128), jnp.float32)

    sc, d2, srcs = _p12(srcp, dstp)
    sc2d = sc.reshape(NP, 1)
    d22d = d2.reshape(NP, 1)

    xlp1, res = _lin1(x_pad, W1, sW, sb.reshape(1, H), sc2d)
    y1 = _spmm(xlp1.reshape(2 * NP, 128), srcs, zeros)
    h1, xlp2 = _statapply(y1.reshape(2, NP, 128), xlp1, sc2d, d22d,
                          b1.reshape(1, H), g1.reshape(1, H),
                          be1.reshape(1, H), res, W2)
    y2 = _spmm(xlp2.reshape(2 * NP, 128), srcs, zeros)
    r = _statfinal(y2.reshape(2, NP, 128), xlp2, sc2d, d22d,
                   b2.reshape(1, H), g2.reshape(1, H), be2.reshape(1, H), h1)
    return r[:N]


# speedup vs baseline: 1.1199x; 1.1199x over previous
import functools

import jax
import jax.numpy as jnp
from jax import lax
from jax.experimental import pallas as pl
from jax.experimental.pallas import tpu as pltpu
from jax.experimental.pallas import tpu_sc as plsc

N = 10000
E = 320000
F_IN = 128
H = 256

NP = 10240
E2 = 327680
ER = E2 // 128
CHUNKS = 40
DUMP = 10008
SELF = 10000
ACC_R = 10112
STRIPE = 632
NB = 10000 // 512 + 1
BLK = 512

_mesh = functools.partial(
    plsc.VectorSubcoreMesh, core_axis_name="c", subcore_axis_name="s")


def _rsqrt16(x):
    i = lax.bitcast_convert_type(x, jnp.int32)
    y = lax.bitcast_convert_type(jnp.int32(0x5F3759DF) - (i >> 1), jnp.float32)
    for _ in range(4):
        y = y * (1.5 - 0.5 * x * y * y)
    return y


def _p12_body(srcp, dstp, s_out, d2_out, srcs_out,
              sidxall, didxall, ones, gs0, gs1, gd0, gd1, nrm0, nrm1,
              sbuf, work, work2, dinv_sh, acc, sem_g, sem_w, sem_n):
    c = lax.axis_index("c")
    s = lax.axis_index("s")
    rt0 = s * 160
    for k in range(8):
        ones[pl.ds(k * 16, 16)] = jnp.full((16,), 1.0, jnp.float32)
    for k in range(40):
        work[pl.ds(k * 16, 16)] = jnp.zeros((16,), jnp.float32)
    pltpu.sync_copy(work, acc.at[pl.ds(s * 640, 640)])
    pltpu.sync_copy(srcp.at[pl.ds(rt0, 160)], sidxall)
    pltpu.sync_copy(dstp.at[pl.ds(rt0, 160)], didxall)
    plsc.subcore_barrier()

    def dchunk(i, carry):
        for j in range(4):
            pltpu.async_copy(ones, acc.at[sidxall.at[i * 4 + j]], sem_g,
                             add=True)
        return carry

    lax.fori_loop(0, 40, dchunk, 0)

    def dwait(i, carry):
        for j in range(4):
            pltpu.make_async_copy(ones, acc.at[sidxall.at[0]], sem_g).wait()
        return carry

    lax.fori_loop(0, 40, dwait, 0)
    plsc.subcore_barrier()
    pltpu.sync_copy(acc.at[pl.ds(s * 640, 640)], work)
    for k in range(40):
        o = pl.ds(k * 16, 16)
        work[o] = _rsqrt16(work[o] + 1.0)
        work2[o] = jnp.zeros((16,), jnp.float32)
    pltpu.sync_copy(work, dinv_sh.at[pl.ds(s * 640, 640)])
    pltpu.sync_copy(work2, acc.at[pl.ds(s * 640, 640)])
    plsc.subcore_barrier()

    def issue(q, gs, gd):
        pltpu.async_copy(dinv_sh.at[sidxall.at[q]], gs, sem_g)
        pltpu.async_copy(dinv_sh.at[didxall.at[q]], gd, sem_g)

    def gwait(q, gs, gd):
        pltpu.make_async_copy(dinv_sh.at[sidxall.at[q]], gs, sem_g).wait()
        pltpu.make_async_copy(dinv_sh.at[didxall.at[q]], gd, sem_g).wait()

    bufs = [(gs0, gd0, nrm0), (gs1, gd1, nrm1)]
    issue(0, gs0, gd0)

    def eblk(k, carry):
        @pl.when(k > 0)
        def _():
            pltpu.make_async_copy(
                sbuf, srcs_out.at[pl.ds(c * ER + rt0, 8)], sem_w).wait()
        for j in range(8):
            q = k * 8 + j
            gs, gd, nrm = bufs[j % 2]
            gs2, gd2, _ = bufs[(j + 1) % 2]
            if j < 7:
                issue(q + 1, gs2, gd2)
            else:
                @pl.when(k < 19)
                def _():
                    issue(q + 1, gs2, gd2)
            gwait(q, gs, gd)
            if j >= 2:
                pltpu.make_async_copy(
                    nrm, acc.at[didxall.at[q]], sem_n).wait()
            else:
                @pl.when(k > 0)
                def _():
                    pltpu.make_async_copy(
                        nrm, acc.at[didxall.at[q]], sem_n).wait()
            for t in range(8):
                o = pl.ds(t * 16, 16)
                sv = sidxall[q, o]
                dv = didxall[q, o]
                eq = sv == dv
                nrm[o] = jnp.where(eq, 0.0, gs[o] * gd[o])
                sbuf[j, o] = (jnp.where(eq, jnp.int32(SELF), sv) + c * NP) | (
                    jnp.where(dv < N, dv, jnp.int32(DUMP)) << 16)
            pltpu.async_copy(nrm, acc.at[didxall.at[q]], sem_n, add=True)
        pltpu.async_copy(
            sbuf, srcs_out.at[pl.ds(c * ER + rt0 + k * 8, 8)], sem_w)
        return carry

    lax.fori_loop(0, 20, eblk, 0)
    for _ in range(2):
        pltpu.make_async_copy(
            nrm0, acc.at[didxall.at[0]], sem_n).wait()
    pltpu.make_async_copy(
        sbuf, srcs_out.at[pl.ds(c * ER + rt0, 8)], sem_w).wait()
    plsc.subcore_barrier()
    base = c * 5120 + s * 320
    pltpu.sync_copy(acc.at[pl.ds(base, 320)], work.at[pl.ds(0, 320)])
    pltpu.sync_copy(dinv_sh.at[pl.ds(base, 320)], work2.at[pl.ds(0, 320)])
    for k in range(20):
        o = pl.ds(k * 16, 16)
        r2 = _rsqrt16(work[o] + 2.0)
        dv = work2[o]
        work[o] = dv * r2
        work2[o] = 2.0 * r2 / dv
    pltpu.sync_copy(work.at[pl.ds(0, 320)], s_out.at[pl.ds(base, 320)])
    pltpu.sync_copy(work2.at[pl.ds(0, 320)], d2_out.at[pl.ds(base, 320)])


_p12 = pl.kernel(
    _p12_body,
    out_type=[
        jax.ShapeDtypeStruct((NP,), jnp.float32),
        jax.ShapeDtypeStruct((NP,), jnp.float32),
        jax.ShapeDtypeStruct((2 * ER, 128), jnp.int32),
    ],
    mesh=_mesh(),
    scratch_types=[
        pltpu.VMEM((160, 128), jnp.int32),
        pltpu.VMEM((160, 128), jnp.int32),
        pltpu.VMEM((128,), jnp.float32),
        pltpu.VMEM((128,), jnp.float32),
        pltpu.VMEM((128,), jnp.float32),
        pltpu.VMEM((128,), jnp.float32),
        pltpu.VMEM((128,), jnp.float32),
        pltpu.VMEM((128,), jnp.float32),
        pltpu.VMEM((128,), jnp.float32),
        pltpu.VMEM((8, 128), jnp.int32),
        pltpu.VMEM((640,), jnp.float32),
        pltpu.VMEM((640,), jnp.float32),
        pltpu.VMEM_SHARED((NP,), jnp.float32),
        pltpu.VMEM_SHARED((NP,), jnp.float32),
        pltpu.SemaphoreType.DMA,
        pltpu.SemaphoreType.DMA,
        pltpu.SemaphoreType.DMA,
    ],
)


def _spmm_body(xlp, pidx_hbm, zeros, y_out,
               pidx, suA, duA, suB, duB, rowsA, rowsB, acc, sem):
    c = lax.axis_index("c")
    s = lax.axis_index("s")
    pltpu.sync_copy(zeros, acc.at[pl.ds(s * STRIPE, STRIPE)])
    plsc.subcore_barrier()

    def unpack(q, su, du):
        for k in range(8):
            o = pl.ds(k * 16, 16)
            p16 = pidx[q, o]
            su[k // 2, pl.ds((k % 2) * 16, 16)] = p16 & jnp.int32(0xFFFF)
            du[0, o] = p16 >> 16

    def gather(su, rows):
        for j in range(4):
            pltpu.async_copy(xlp.at[su.at[j]], rows.at[pl.ds(j * 32, 32)], sem)

    def gwait(su, rows):
        for j in range(4):
            pltpu.make_async_copy(
                xlp.at[su.at[j]], rows.at[pl.ds(j * 32, 32)], sem).wait()

    def scat(rows, du):
        pltpu.sync_copy(rows, acc.at[du.at[0]], add=True)

    for p in range(2):
        pltpu.sync_copy(pidx_hbm.at[pl.ds(c * ER + s * 160 + p * 80, 80)], pidx)
        unpack(0, suA, duA)
        gather(suA, rowsA)

        def it(k, carry):
            q = 2 * k
            unpack(q + 1, suB, duB)
            gather(suB, rowsB)
            gwait(suA, rowsA)
            scat(rowsA, duA)
            unpack(q + 2, suA, duA)
            gather(suA, rowsA)
            gwait(suB, rowsB)
            scat(rowsB, duB)
            return carry

        lax.fori_loop(0, 39, it, 0)
        unpack(79, suB, duB)
        gather(suB, rowsB)
        gwait(suA, rowsA)
        scat(rowsA, duA)
        gwait(suB, rowsB)
        scat(rowsB, duB)

    plsc.subcore_barrier()
    pltpu.sync_copy(acc.at[pl.ds(s * STRIPE, STRIPE)],
                    y_out.at[pl.ds(c * NP + s * STRIPE, STRIPE)])


_spmm = pl.kernel(
    _spmm_body,
    out_type=jax.ShapeDtypeStruct((2 * NP, 128), jnp.float32),
    mesh=_mesh(),
    scratch_types=[
        pltpu.VMEM((80, 128), jnp.int32),
        pltpu.VMEM((4, 32), jnp.int32),
        pltpu.VMEM((1, 128), jnp.int32),
        pltpu.VMEM((4, 32), jnp.int32),
        pltpu.VMEM((1, 128), jnp.int32),
        pltpu.VMEM((128, 128), jnp.float32),
        pltpu.VMEM((128, 128), jnp.float32),
        pltpu.VMEM_SHARED((ACC_R, 128), jnp.float32),
        pltpu.SemaphoreType.DMA,
    ],
)


def _lin1_body(x_ref, w1_ref, sw_ref, sb_ref, sc_ref, xlp_ref, res_ref):
    x = x_ref[...]
    sc = sc_ref[...]
    mm = lax.dot_general(x, w1_ref[...], (((1,), (1,)), ((), ())),
                         preferred_element_type=jnp.float32)
    res_ref[...] = lax.dot_general(x, sw_ref[...], (((1,), (1,)), ((), ())),
                                   preferred_element_type=jnp.float32) + sb_ref[...]
    xlp_ref[0] = sc * mm[:, 0:128]
    xlp_ref[1] = sc * mm[:, 128:256]


def _stat_body(y_ref, xlp_ref, sc_ref, d2_ref, b_ref, out_ref, sums_ref):
    pid = pl.program_id(0)
    sc = sc_ref[...]
    d2 = d2_ref[...]
    b = b_ref[...]
    o_l = sc * y_ref[0] + d2 * xlp_ref[0] + b[:, 0:128]
    o_r = sc * y_ref[1] + d2 * xlp_ref[1] + b[:, 128:256]
    out_ref[:, 0:128] = o_l
    out_ref[:, 128:256] = o_r
    rows = lax.broadcasted_iota(jnp.int32, (BLK, 1), 0) + pid * BLK
    mask = rows < N
    ml = jnp.where(mask, o_l, 0.0)
    mr = jnp.where(mask, o_r, 0.0)
    p = jnp.concatenate(
        [jnp.sum(ml, axis=0, keepdims=True),
         jnp.sum(mr, axis=0, keepdims=True)], axis=1)
    q = jnp.concatenate(
        [jnp.sum(ml * ml, axis=0, keepdims=True),
         jnp.sum(mr * mr, axis=0, keepdims=True)], axis=1)
    ps = jnp.concatenate([p, q], axis=0)

    @pl.when(pid == 0)
    def _():
        sums_ref[...] = ps

    @pl.when(pid > 0)
    def _():
        sums_ref[...] += ps


def _applylin_body(out_ref, sums_ref, g_ref, be_ref, res_ref, sc_ref, w2_ref,
                   h1_ref, xlp2_ref):
    pid = pl.program_id(0)
    mu = sums_ref[0:1] * (1.0 / N)
    var = sums_ref[1:2] * (1.0 / N) - mu * mu
    rs = lax.rsqrt(var + 1e-5)
    bn = g_ref[...] * (out_ref[...] - mu) * rs + be_ref[...]
    h1 = jnp.maximum(jnp.maximum(bn, 0.0) + res_ref[...], 0.0)
    rows = lax.broadcasted_iota(jnp.int32, (BLK, 1), 0) + pid * BLK
    h1 = jnp.where(rows < N, h1, 0.0)
    h1_ref[...] = h1
    mm = lax.dot_general(h1, w2_ref[...], (((1,), (1,)), ((), ())),
                         preferred_element_type=jnp.float32)
    sc = sc_ref[...]
    xlp2_ref[0] = sc * mm[:, 0:128]
    xlp2_ref[1] = sc * mm[:, 128:256]


def _final_body(out_ref, sums_ref, g_ref, be_ref, h1_ref, r_ref):
    mu = sums_ref[0:1] * (1.0 / N)
    var = sums_ref[1:2] * (1.0 / N) - mu * mu
    rs = lax.rsqrt(var + 1e-5)
    bn = g_ref[...] * (out_ref[...] - mu) * rs + be_ref[...]
    h2 = jnp.maximum(jnp.maximum(bn, 0.0) + h1_ref[...], 0.0)
    nrm = jnp.sqrt(jnp.sum(h2 * h2, axis=1, keepdims=True))
    r_ref[...] = h2 / jnp.maximum(nrm, 1e-12)


def _row_spec(w):
    return pl.BlockSpec((BLK, w), lambda i: (i, 0))


def _half_spec():
    return pl.BlockSpec((2, BLK, 128), lambda i: (0, i, 0))


def _full_spec(shape):
    return pl.BlockSpec(shape, lambda i: tuple(0 for _ in shape))


_lin1 = pl.pallas_call(
    _lin1_body,
    grid=(NB,),
    in_specs=[_row_spec(128), _full_spec((H, F_IN)), _full_spec((H, F_IN)),
              _full_spec((1, H)), _row_spec(1)],
    out_specs=[_half_spec(), _row_spec(H)],
    out_shape=[jax.ShapeDtypeStruct((2, NP, 128), jnp.float32),
               jax.ShapeDtypeStruct((NP, H), jnp.float32)],
)

_stat = pl.pallas_call(
    _stat_body,
    grid=(NB,),
    in_specs=[_half_spec(), _half_spec(), _row_spec(1), _row_spec(1),
              _full_spec((1, H))],
    out_specs=[_row_spec(H), _full_spec((2, H))],
    out_shape=[jax.ShapeDtypeStruct((NP, H), jnp.float32),
               jax.ShapeDtypeStruct((2, H), jnp.float32)],
)

_applylin = pl.pallas_call(
    _applylin_body,
    grid=(NB,),
    in_specs=[_row_spec(H), _full_spec((2, H)), _full_spec((1, H)),
              _full_spec((1, H)), _row_spec(H), _row_spec(1),
              _full_spec((H, H))],
    out_specs=[_row_spec(H), _half_spec()],
    out_shape=[jax.ShapeDtypeStruct((NP, H), jnp.float32),
               jax.ShapeDtypeStruct((2, NP, 128), jnp.float32)],
)

_final = pl.pallas_call(
    _final_body,
    grid=(NB,),
    in_specs=[_row_spec(H), _full_spec((2, H)), _full_spec((1, H)),
              _full_spec((1, H)), _row_spec(H)],
    out_specs=_row_spec(H),
    out_shape=jax.ShapeDtypeStruct((NP, H), jnp.float32),
)


def kernel(x, edge_index, W1, b1, g1, be1, sW, sb, W2, b2, g2, be2):
    pad = jnp.full((E2 - E,), DUMP, jnp.int32)
    srcp = jnp.concatenate([edge_index[0], pad]).reshape(ER, 128)
    dstp = jnp.concatenate([edge_index[1], pad]).reshape(ER, 128)
    x_pad = jnp.pad(x, ((0, NP - N), (0, 0)))
    zeros = jnp.zeros((STRIPE, 128), jnp.float32)

    sc, d2, srcs = _p12(srcp, dstp)
    sc2d = sc.reshape(NP, 1)
    d22d = d2.reshape(NP, 1)

    xlp1, res = _lin1(x_pad, W1, sW, sb.reshape(1, H), sc2d)
    y1 = _spmm(xlp1.reshape(2 * NP, 128), srcs, zeros)
    out1, sums1 = _stat(y1.reshape(2, NP, 128), xlp1, sc2d, d22d,
                        b1.reshape(1, H))
    h1, xlp2 = _applylin(out1, sums1, g1.reshape(1, H), be1.reshape(1, H),
                         res, sc2d, W2)
    y2 = _spmm(xlp2.reshape(2 * NP, 128), srcs, zeros)
    out2, sums2 = _stat(y2.reshape(2, NP, 128), xlp2, sc2d, d22d,
                        b2.reshape(1, H))
    r = _final(out2, sums2, g2.reshape(1, H), be2.reshape(1, H), h1)
    return r[:N]
